# baseline (speedup 1.0000x reference)
import functools

import jax
import jax.numpy as jnp
from jax import lax
from jax.experimental import pallas as pl
from jax.experimental.pallas import tpu as pltpu
from jax.experimental.pallas import tpu_sc as plsc

S, H, I, E, K = 2048, 1024, 1024, 64, 2
NA = S * K
TB = 128
NT = NA // TB + E
NPAD = (NT + 1) * TB
CH = 512

NC, NS = 2, 16
NW = NC * NS
L = 16
K2_C, K2_W = 4, 32
K4_C, K4_W = 4, 16


def _router_body(x_ref, rw_ref, logits_ref, pos_ref, w_ref, sp_ref):
    x = x_ref[...]
    logits = jnp.dot(x, rw_ref[...], preferred_element_type=jnp.float32)
    logits_ref[...] = logits

    iota_e = lax.broadcasted_iota(jnp.int32, (CH, E), 1)
    r_i = lax.broadcasted_iota(jnp.int32, (CH, CH), 0)
    c_i = lax.broadcasted_iota(jnp.int32, (CH, CH), 1)
    tril = (r_i > c_i).astype(jnp.float32)

    e1s, e2s, w1s, w2s = [], [], [], []
    for c in range(S // CH):
        lg = logits[c * CH:(c + 1) * CH, :]
        mx = jnp.max(lg, axis=1, keepdims=True)
        ex = jnp.exp(lg - mx)
        p = ex / jnp.sum(ex, axis=1, keepdims=True)
        m1 = jnp.max(p, axis=1, keepdims=True)
        e1 = jnp.min(jnp.where(p == m1, iota_e, E), axis=1, keepdims=True)
        p2 = jnp.where(iota_e == e1, -1.0, p)
        m2 = jnp.max(p2, axis=1, keepdims=True)
        e2 = jnp.min(jnp.where(p2 == m2, iota_e, E), axis=1, keepdims=True)
        sw = m1 + m2
        e1s.append(e1); e2s.append(e2)
        w1s.append(m1 / sw); w2s.append(m2 / sw)

    carry = jnp.zeros((1, E), jnp.float32)
    onehots, ranks = [], []
    for es in (e1s, e2s):
        for c in range(S // CH):
            a = (iota_e == es[c]).astype(jnp.float32)
            rk_full = jnp.dot(tril, a, preferred_element_type=jnp.float32) + carry
            ranks.append(jnp.sum(rk_full * a, axis=1, keepdims=True))
            onehots.append(a)
            carry = carry + jnp.sum(a, axis=0, keepdims=True)

    counts_i = carry.astype(jnp.int32)
    tiles = (((counts_i + (TB - 1)) // TB)).astype(jnp.float32)
    tiles_ext = jnp.concatenate([tiles, jnp.zeros((1, 128 - E), jnp.float32)],
                                axis=1)
    u_r = lax.broadcasted_iota(jnp.int32, (128, 128), 0)
    u_c = lax.broadcasted_iota(jnp.int32, (128, 128), 1)
    upper = (u_r < u_c).astype(jnp.float32)
    toff = jnp.dot(tiles_ext, upper, preferred_element_type=jnp.float32)
    offsets = toff * float(TB)
    tends = toff[:, :E] + tiles
    n_tiles = jnp.sum(tiles, axis=1, keepdims=True)

    jj = lax.broadcasted_iota(jnp.int32, (128, E), 0).astype(jnp.float32)
    texp = jnp.sum((tends <= jj).astype(jnp.float32), axis=1, keepdims=True)
    texp = jnp.minimum(texp, float(E - 1))
    jcol = lax.broadcasted_iota(jnp.int32, (128, 1), 0).astype(jnp.float32)
    trow = jnp.where(jcol < n_tiles, jcol, float(NT))
    sp_ref[:, 0:1] = texp.astype(jnp.int32)
    sp_ref[:, 1:2] = trow.astype(jnp.int32)

    idx = 0
    off_row = offsets[:, :E]
    for slot in range(K):
        for c in range(S // CH):
            a = onehots[idx]
            base = jnp.sum(a * off_row, axis=1, keepdims=True)
            posv = (base + ranks[idx]).astype(jnp.int32)
            pos_ref[pl.ds(c * CH, CH), pl.ds(slot, 1)] = posv
            wv = w1s[c] if slot == 0 else w2s[c]
            w_ref[pl.ds(c * CH, CH), pl.ds(slot, 1)] = wv
            idx += 1


def _routing_call(x2d, router_w):
    return pl.pallas_call(
        _router_body,
        out_shape=[
            jax.ShapeDtypeStruct((S, E), jnp.float32),
            jax.ShapeDtypeStruct((S, K), jnp.int32),
            jax.ShapeDtypeStruct((S, K), jnp.float32),
            jax.ShapeDtypeStruct((128, 2), jnp.int32),
        ],
    )(x2d, router_w)


def _dispatch_body(x_hbm, sidx_hbm, xs_hbm, sidx_v, buf0, buf1, gsem, ssem):
    wid = lax.axis_index("s") * NC + lax.axis_index("c")
    pltpu.sync_copy(sidx_hbm.at[wid], sidx_v)
    base = wid * (K2_C * K2_W)
    bufs = (buf0, buf1)

    def gather(c):
        src = (base + c * K2_W) % S
        return pltpu.async_copy(x_hbm.at[pl.ds(src, K2_W)], bufs[c % 2], gsem)

    gcp = gather(0)
    scp = None
    for c in range(K2_C):
        gcp.wait()
        if scp is not None:
            scp.wait()
        gcp = gather(c + 1) if c + 1 < K2_C else None
        scp = pltpu.async_copy(bufs[c % 2], xs_hbm.at[sidx_v.at[c]], ssem)
    scp.wait()


def _dispatch_call(x2d, sidx):
    mesh = plsc.VectorSubcoreMesh(core_axis_name="c", subcore_axis_name="s")
    f = functools.partial(
        pl.kernel,
        mesh=mesh,
        out_type=jax.ShapeDtypeStruct((NPAD, H), jnp.float32),
        scratch_types=[
            pltpu.VMEM((K2_C, K2_W), jnp.int32),
            pltpu.VMEM((K2_W, H), jnp.float32),
            pltpu.VMEM((K2_W, H), jnp.float32),
            pltpu.SemaphoreType.DMA,
            pltpu.SemaphoreType.DMA,
        ],
    )(_dispatch_body)
    return f(x2d, sidx)


def _expert_body(sp_ref, xs_ref, wg_ref, wi_ref, wo_ref, ys_ref):
    xt = xs_ref[...].astype(jnp.bfloat16)
    g = jnp.dot(xt, wg_ref[0].astype(jnp.bfloat16),
                preferred_element_type=jnp.float32)
    g = g * jax.nn.sigmoid(g)
    it = jnp.dot(xt, wi_ref[0].astype(jnp.bfloat16),
                 preferred_element_type=jnp.float32)
    ys_ref[...] = jnp.dot((g * it).astype(jnp.bfloat16),
                          wo_ref[0].astype(jnp.bfloat16),
                          preferred_element_type=jnp.float32)


def _expert_call(sp, xs, w_gate, w_inter, w_out):
    grid_spec = pltpu.PrefetchScalarGridSpec(
        num_scalar_prefetch=1,
        grid=(NT,),
        in_specs=[
            pl.BlockSpec((TB, H), lambda j, sp: (sp[j, 1], 0)),
            pl.BlockSpec((1, H, I), lambda j, sp: (sp[j, 0], 0, 0)),
            pl.BlockSpec((1, H, I), lambda j, sp: (sp[j, 0], 0, 0)),
            pl.BlockSpec((1, I, H), lambda j, sp: (sp[j, 0], 0, 0)),
        ],
        out_specs=pl.BlockSpec((TB, H), lambda j, sp: (sp[j, 1], 0)),
    )
    return pl.pallas_call(
        _expert_body,
        grid_spec=grid_spec,
        out_shape=jax.ShapeDtypeStruct((NPAD, H), jnp.float32),
        compiler_params=pltpu.CompilerParams(
            vmem_limit_bytes=100 * 1024 * 1024,
        ),
    )(sp, xs, w_gate, w_inter, w_out)


def _combine_body(ys_hbm, pa_hbm, pb_hbm, w1_hbm, w2_hbm, out_hbm,
                  pa_v, pb_v, w1_v, w2_v, b1a, b1b, b2a, b2b, oba, obb,
                  gsem1, gsem2, osem1, osem2):
    wid = lax.axis_index("s") * NC + lax.axis_index("c")
    pltpu.sync_copy(pa_hbm.at[wid], pa_v)
    pltpu.sync_copy(pb_hbm.at[wid], pb_v)
    pltpu.sync_copy(w1_hbm.at[wid], w1_v)
    pltpu.sync_copy(w2_hbm.at[wid], w2_v)
    b1s, b2s, obs = (b1a, b1b), (b2a, b2b), (oba, obb)
    osems = (osem1, osem2)

    def gathers(c):
        return (pltpu.async_copy(ys_hbm.at[pa_v.at[c]], b1s[c % 2], gsem1),
                pltpu.async_copy(ys_hbm.at[pb_v.at[c]], b2s[c % 2], gsem2))

    gcp = gathers(0)
    ocp = [None, None]
    for c in range(K4_C):
        gcp[0].wait()
        gcp[1].wait()
        gcp = gathers(c + 1) if c + 1 < K4_C else gcp
        if ocp[c % 2] is not None:
            ocp[c % 2].wait()
        b1, b2, ob = b1s[c % 2], b2s[c % 2], obs[c % 2]
        for r in range(K4_W):
            w1vec = w1_v[c, r, :]
            w2vec = w2_v[c, r, :]

            @pl.loop(0, H // (4 * L))
            def _(j):
                for u in range(4):
                    sl = pl.ds(pl.multiple_of(j * 4 * L + u * L, L), L)
                    ob[r, sl] = b1[r, sl] * w1vec + b2[r, sl] * w2vec

        ocp[c % 2] = pltpu.async_copy(
            ob, out_hbm.at[pl.ds(wid * (K4_C * K4_W) + c * K4_W, K4_W)],
            osems[c % 2])
    ocp[0].wait()
    ocp[1].wait()


def _combine_call(ys, pa, pb, w1b, w2b):
    mesh = plsc.VectorSubcoreMesh(core_axis_name="c", subcore_axis_name="s")
    f = functools.partial(
        pl.kernel,
        mesh=mesh,
        out_type=jax.ShapeDtypeStruct((S, H), jnp.float32),
        scratch_types=[
            pltpu.VMEM((K4_C, K4_W), jnp.int32),
            pltpu.VMEM((K4_C, K4_W), jnp.int32),
            pltpu.VMEM((K4_C, K4_W, L), jnp.float32),
            pltpu.VMEM((K4_C, K4_W, L), jnp.float32),
            pltpu.VMEM((K4_W, H), jnp.float32),
            pltpu.VMEM((K4_W, H), jnp.float32),
            pltpu.VMEM((K4_W, H), jnp.float32),
            pltpu.VMEM((K4_W, H), jnp.float32),
            pltpu.VMEM((K4_W, H), jnp.float32),
            pltpu.VMEM((K4_W, H), jnp.float32),
            pltpu.SemaphoreType.DMA,
            pltpu.SemaphoreType.DMA,
            pltpu.SemaphoreType.DMA,
            pltpu.SemaphoreType.DMA,
        ],
    )(_combine_body)
    return f(ys, pa, pb, w1b, w2b)


def kernel(hidden_states, router_w, w_gate, w_inter, w_out):
    b, s, h = hidden_states.shape
    x2d = hidden_states.reshape(s * b, h)

    logits, posw, ww, sp = _routing_call(x2d, router_w)

    sidx = jnp.concatenate([posw[:, 0], posw[:, 1]]).reshape(NW, K2_C, K2_W)
    xs = _dispatch_call(x2d, sidx)

    ys = _expert_call(sp, xs, w_gate, w_inter, w_out)

    pa = posw[:, 0].reshape(NW, K4_C, K4_W)
    pb = posw[:, 1].reshape(NW, K4_C, K4_W)
    w1b = jnp.broadcast_to(ww[:, 0:1], (S, L)).reshape(NW, K4_C, K4_W, L)
    w2b = jnp.broadcast_to(ww[:, 1:2], (S, L)).reshape(NW, K4_C, K4_W, L)
    final2d = _combine_call(ys, pa, pb, w1b, w2b)

    return final2d.reshape(b, s, h), logits


# speedup vs baseline: 1.0048x; 1.0048x over previous
import functools

import jax
import jax.numpy as jnp
from jax import lax
from jax.experimental import pallas as pl
from jax.experimental.pallas import tpu as pltpu
from jax.experimental.pallas import tpu_sc as plsc

S, H, I, E, K = 2048, 1024, 1024, 64, 2
NA = S * K
TB = 128
NT = NA // TB + E
NPAD = (NT + 1) * TB
CH = 512

NC, NS = 2, 16
NW = NC * NS
L = 16
K2_C, K2_W = 4, 32
K4_C, K4_W = 4, 16


def _router_body(x_ref, rw_ref, logits_ref, pos_ref, w_ref, sp_ref):
    x = x_ref[...]
    logits = jnp.dot(x, rw_ref[...], preferred_element_type=jnp.float32)
    logits_ref[...] = logits

    iota_e = lax.broadcasted_iota(jnp.int32, (CH, E), 1)
    r_i = lax.broadcasted_iota(jnp.int32, (CH, CH), 0)
    c_i = lax.broadcasted_iota(jnp.int32, (CH, CH), 1)
    tril = (r_i > c_i).astype(jnp.float32)

    e1s, e2s, w1s, w2s = [], [], [], []
    for c in range(S // CH):
        lg = logits[c * CH:(c + 1) * CH, :]
        mx = jnp.max(lg, axis=1, keepdims=True)
        ex = jnp.exp(lg - mx)
        p = ex / jnp.sum(ex, axis=1, keepdims=True)
        m1 = jnp.max(p, axis=1, keepdims=True)
        e1 = jnp.min(jnp.where(p == m1, iota_e, E), axis=1, keepdims=True)
        p2 = jnp.where(iota_e == e1, -1.0, p)
        m2 = jnp.max(p2, axis=1, keepdims=True)
        e2 = jnp.min(jnp.where(p2 == m2, iota_e, E), axis=1, keepdims=True)
        sw = m1 + m2
        e1s.append(e1); e2s.append(e2)
        w1s.append(m1 / sw); w2s.append(m2 / sw)

    carry = jnp.zeros((1, E), jnp.float32)
    onehots, ranks = [], []
    for es in (e1s, e2s):
        for c in range(S // CH):
            a = (iota_e == es[c]).astype(jnp.float32)
            rk_full = jnp.dot(tril, a, preferred_element_type=jnp.float32) + carry
            ranks.append(jnp.sum(rk_full * a, axis=1, keepdims=True))
            onehots.append(a)
            carry = carry + jnp.sum(a, axis=0, keepdims=True)

    counts_i = carry.astype(jnp.int32)
    tiles = (((counts_i + (TB - 1)) // TB)).astype(jnp.float32)
    tiles_ext = jnp.concatenate([tiles, jnp.zeros((1, 128 - E), jnp.float32)],
                                axis=1)
    u_r = lax.broadcasted_iota(jnp.int32, (128, 128), 0)
    u_c = lax.broadcasted_iota(jnp.int32, (128, 128), 1)
    upper = (u_r < u_c).astype(jnp.float32)
    toff = jnp.dot(tiles_ext, upper, preferred_element_type=jnp.float32)
    offsets = toff * float(TB)
    tends = toff[:, :E] + tiles
    n_tiles = jnp.sum(tiles, axis=1, keepdims=True)

    jj = lax.broadcasted_iota(jnp.int32, (128, E), 0).astype(jnp.float32)
    texp = jnp.sum((tends <= jj).astype(jnp.float32), axis=1, keepdims=True)
    texp = jnp.minimum(texp, float(E - 1))
    jcol = lax.broadcasted_iota(jnp.int32, (128, 1), 0).astype(jnp.float32)
    trow = jnp.where(jcol < n_tiles, jcol, float(NT))
    sp_ref[:, 0:1] = texp.astype(jnp.int32)
    sp_ref[:, 1:2] = trow.astype(jnp.int32)

    idx = 0
    off_row = offsets[:, :E]
    for slot in range(K):
        for c in range(S // CH):
            a = onehots[idx]
            base = jnp.sum(a * off_row, axis=1, keepdims=True)
            posv = (base + ranks[idx]).astype(jnp.int32)
            pos_ref[pl.ds(c * CH, CH), pl.ds(slot, 1)] = posv
            wv = w1s[c] if slot == 0 else w2s[c]
            w_ref[pl.ds(c * CH, CH), pl.ds(slot, 1)] = wv
            idx += 1


def _routing_call(x2d, router_w):
    return pl.pallas_call(
        _router_body,
        out_shape=[
            jax.ShapeDtypeStruct((S, E), jnp.float32),
            jax.ShapeDtypeStruct((S, K), jnp.int32),
            jax.ShapeDtypeStruct((S, K), jnp.float32),
            jax.ShapeDtypeStruct((128, 2), jnp.int32),
        ],
    )(x2d, router_w)


def _dispatch_body(x_hbm, sidx_hbm, xs_hbm, sidx_v, buf0, buf1, gsem, ssem):
    wid = lax.axis_index("s") * NC + lax.axis_index("c")
    pltpu.sync_copy(sidx_hbm.at[wid], sidx_v)
    base = wid * (K2_C * K2_W)
    bufs = (buf0, buf1)

    def gather(c):
        src = (base + c * K2_W) % S
        return pltpu.async_copy(x_hbm.at[pl.ds(src, K2_W)], bufs[c % 2], gsem)

    gcp = gather(0)
    scp = None
    for c in range(K2_C):
        gcp.wait()
        if scp is not None:
            scp.wait()
        gcp = gather(c + 1) if c + 1 < K2_C else None
        scp = pltpu.async_copy(bufs[c % 2], xs_hbm.at[sidx_v.at[c]], ssem)
    scp.wait()


def _dispatch_call(x2d, sidx):
    mesh = plsc.VectorSubcoreMesh(core_axis_name="c", subcore_axis_name="s")
    f = functools.partial(
        pl.kernel,
        mesh=mesh,
        out_type=jax.ShapeDtypeStruct((NPAD, H), jnp.float32),
        scratch_types=[
            pltpu.VMEM((K2_C, K2_W), jnp.int32),
            pltpu.VMEM((K2_W, H), jnp.float32),
            pltpu.VMEM((K2_W, H), jnp.float32),
            pltpu.SemaphoreType.DMA,
            pltpu.SemaphoreType.DMA,
        ],
    )(_dispatch_body)
    return f(x2d, sidx)


def _expert_body(sp_ref, xs_ref, wg_ref, wi_ref, wo_ref, ys_ref):
    xt = xs_ref[...].astype(jnp.bfloat16)
    g = jnp.dot(xt, wg_ref[0].astype(jnp.bfloat16),
                preferred_element_type=jnp.float32)
    g = g * jax.nn.sigmoid(g)
    it = jnp.dot(xt, wi_ref[0].astype(jnp.bfloat16),
                 preferred_element_type=jnp.float32)
    ys_ref[...] = jnp.dot((g * it).astype(jnp.bfloat16),
                          wo_ref[0].astype(jnp.bfloat16),
                          preferred_element_type=jnp.float32)


def _expert_call(sp, xs, w_gate, w_inter, w_out):
    grid_spec = pltpu.PrefetchScalarGridSpec(
        num_scalar_prefetch=1,
        grid=(NT,),
        in_specs=[
            pl.BlockSpec((TB, H), lambda j, sp: (sp[j, 1], 0)),
            pl.BlockSpec((1, H, I), lambda j, sp: (sp[j, 0], 0, 0)),
            pl.BlockSpec((1, H, I), lambda j, sp: (sp[j, 0], 0, 0)),
            pl.BlockSpec((1, I, H), lambda j, sp: (sp[j, 0], 0, 0)),
        ],
        out_specs=pl.BlockSpec((TB, H), lambda j, sp: (sp[j, 1], 0)),
    )
    return pl.pallas_call(
        _expert_body,
        grid_spec=grid_spec,
        out_shape=jax.ShapeDtypeStruct((NPAD, H), jnp.float32),
        compiler_params=pltpu.CompilerParams(
            vmem_limit_bytes=100 * 1024 * 1024,
        ),
    )(sp, xs, w_gate, w_inter, w_out)


def _combine_body(ys_hbm, pa_hbm, pb_hbm, w1_hbm, w2_hbm, out_hbm,
                  pa_v, pb_v, w1_v, w2_v, b1a, b1b, b2a, b2b, oba, obb,
                  gsem1, gsem2, osem1, osem2):
    wid = lax.axis_index("s") * NC + lax.axis_index("c")
    pltpu.sync_copy(pa_hbm.at[wid], pa_v)
    pltpu.sync_copy(pb_hbm.at[wid], pb_v)
    pltpu.sync_copy(w1_hbm.at[wid], w1_v)
    pltpu.sync_copy(w2_hbm.at[wid], w2_v)
    b1s, b2s, obs = (b1a, b1b), (b2a, b2b), (oba, obb)
    osems = (osem1, osem2)

    def gathers(c):
        return (pltpu.async_copy(ys_hbm.at[pa_v.at[c]], b1s[c % 2], gsem1),
                pltpu.async_copy(ys_hbm.at[pb_v.at[c]], b2s[c % 2], gsem2))

    gcp = gathers(0)
    ocp = [None, None]
    for c in range(K4_C):
        gcp[0].wait()
        gcp[1].wait()
        gcp = gathers(c + 1) if c + 1 < K4_C else gcp
        if ocp[c % 2] is not None:
            ocp[c % 2].wait()
        b1, b2, ob = b1s[c % 2], b2s[c % 2], obs[c % 2]
        for r in range(K4_W):
            w1vec = w1_v[c, r, :]
            w2vec = w2_v[c, r, :]

            @pl.loop(0, H // L)
            def _(j):
                sl = pl.ds(pl.multiple_of(j * L, L), L)
                ob[r, sl] = b1[r, sl] * w1vec + b2[r, sl] * w2vec

        ocp[c % 2] = pltpu.async_copy(
            ob, out_hbm.at[pl.ds(wid * (K4_C * K4_W) + c * K4_W, K4_W)],
            osems[c % 2])
    ocp[0].wait()
    ocp[1].wait()


def _combine_call(ys, pa, pb, w1b, w2b):
    mesh = plsc.VectorSubcoreMesh(core_axis_name="c", subcore_axis_name="s")
    f = functools.partial(
        pl.kernel,
        mesh=mesh,
        out_type=jax.ShapeDtypeStruct((S, H), jnp.float32),
        scratch_types=[
            pltpu.VMEM((K4_C, K4_W), jnp.int32),
            pltpu.VMEM((K4_C, K4_W), jnp.int32),
            pltpu.VMEM((K4_C, K4_W, L), jnp.float32),
            pltpu.VMEM((K4_C, K4_W, L), jnp.float32),
            pltpu.VMEM((K4_W, H), jnp.float32),
            pltpu.VMEM((K4_W, H), jnp.float32),
            pltpu.VMEM((K4_W, H), jnp.float32),
            pltpu.VMEM((K4_W, H), jnp.float32),
            pltpu.VMEM((K4_W, H), jnp.float32),
            pltpu.VMEM((K4_W, H), jnp.float32),
            pltpu.SemaphoreType.DMA,
            pltpu.SemaphoreType.DMA,
            pltpu.SemaphoreType.DMA,
            pltpu.SemaphoreType.DMA,
        ],
    )(_combine_body)
    return f(ys, pa, pb, w1b, w2b)


def kernel(hidden_states, router_w, w_gate, w_inter, w_out):
    b, s, h = hidden_states.shape
    x2d = hidden_states.reshape(s * b, h)

    logits, posw, ww, sp = _routing_call(x2d, router_w)

    sidx = jnp.concatenate([posw[:, 0], posw[:, 1]]).reshape(NW, K2_C, K2_W)
    xs = _dispatch_call(x2d, sidx)

    ys = _expert_call(sp, xs, w_gate, w_inter, w_out)

    pa = posw[:, 0].reshape(NW, K4_C, K4_W)
    pb = posw[:, 1].reshape(NW, K4_C, K4_W)
    w1b = jnp.broadcast_to(ww[:, 0:1], (S, L)).reshape(NW, K4_C, K4_W, L)
    w2b = jnp.broadcast_to(ww[:, 1:2], (S, L)).reshape(NW, K4_C, K4_W, L)
    final2d = _combine_call(ys, pa, pb, w1b, w2b)

    return final2d.reshape(b, s, h), logits
